# baseline (speedup 1.0000x reference)
import functools

import jax
import jax.numpy as jnp
from jax.experimental import pallas as pl
from jax.experimental.pallas import tpu as pltpu

S = 20
N = 128
HID = 32
G = 4
NB = N // G
P4 = N * NB
BA = 32
PB = BA * NB


def _split4(x):
    return jnp.concatenate([x[0:32], x[32:64], x[64:96], x[96:128]], axis=1)


def _bc_rows(x):
    return jnp.broadcast_to(x[:, None, :], (BA, NB, 128)).reshape(PB, 128)


def _bc_cols(x):
    return jnp.broadcast_to(x[None, :, :], (BA, NB, 128)).reshape(PB, 128)


def _body(corr_ref, dc_ref, ni_ref, hs_ref, mblk_ref,
          w4blk_ref, w2blk_ref, relb_ref, rellnw_ref, rellnb_ref,
          ngWr_ref, ngWh_ref, ngWn_ref, ngb_ref, nglnw_ref, nglnb_ref,
          warWr_ref, warWh_ref, warWn_ref, warb_ref, rup_ref,
          wW_ref, wb_ref, wlnw_ref, wlnb_ref,
          out_ref):
    mblk = mblk_ref[...]

    def ln_packed(x, w, b, eps=1e-05):
        u = jnp.dot(x, mblk, preferred_element_type=jnp.float32)
        d = x - u
        s = jnp.dot(d * d, mblk, preferred_element_type=jnp.float32)
        return d * (w * jax.lax.rsqrt(s + eps)) + b

    h_s = hs_ref[0]
    ia = pl.program_id(1)
    h_a = hs_ref[0, pl.ds(ia * BA, BA), :]

    r_lin = (jnp.dot(corr_ref[...], w4blk_ref[...],
                     preferred_element_type=jnp.float32)
             + jnp.dot(dc_ref[0], w2blk_ref[...],
                       preferred_element_type=jnp.float32)
             + relb_ref[...])
    r_t = jax.nn.relu(ln_packed(r_lin, rellnw_ref[...], rellnb_ref[...]))

    hh = jnp.dot(h_a, ngWh_ref[...], preferred_element_type=jnp.float32)
    hn = jnp.dot(h_s, ngWn_ref[...], preferred_element_type=jnp.float32)
    hh_bc = _bc_rows(jnp.concatenate([hh, hh, hh, hh], axis=1))
    hn_bc = _bc_cols(_split4(hn))
    ng_lin = (jnp.dot(r_t, ngWr_ref[...], preferred_element_type=jnp.float32)
              + hh_bc + hn_bc + ngb_ref[...])
    nGate = jax.nn.sigmoid(ln_packed(ng_lin, nglnw_ref[...], nglnb_ref[...]))

    wh = jnp.dot(h_a, warWh_ref[...], preferred_element_type=jnp.float32)
    wn = jnp.dot(h_s, warWn_ref[...], preferred_element_type=jnp.float32)
    z4 = (jnp.dot(r_t, warWr_ref[...], preferred_element_type=jnp.float32)
          .reshape(BA, NB, G)
          + wh.reshape(BA, 1, 1) + _split4(wn)[None, :, :] + warb_ref[...])
    tt3 = jnp.swapaxes(jax.nn.relu(z4), 1, 2)
    tt_nn = jnp.concatenate(
        [tt3[:, 0, :], tt3[:, 1, :], tt3[:, 2, :], tt3[:, 3, :]], axis=1)

    mask = ni_ref[...] > 0
    pos_in = jnp.where(mask & (tt_nn != 0.0), tt_nn, -10000.0)
    pos = jax.nn.softmax(pos_in, axis=-1)
    coef = jnp.where(mask, pos, 0.0)

    c34 = jnp.concatenate(
        [coef[:, 0:32].reshape(BA, 1, NB), coef[:, 32:64].reshape(BA, 1, NB),
         coef[:, 64:96].reshape(BA, 1, NB), coef[:, 96:128].reshape(BA, 1, NB)],
        axis=1)
    coef4 = jnp.swapaxes(c34, 1, 2).reshape(PB, G)
    coef_p = jnp.dot(coef4, rup_ref[...],
                     preferred_element_type=jnp.float32)

    nei_p = _bc_cols(_split4(h_s))
    hm = nei_p * nGate * coef_p
    hsum = hm.reshape(BA, NB, 128).sum(axis=1)
    h_sum_in = (hsum[:, 0:32] + hsum[:, 32:64]
                + hsum[:, 64:96] + hsum[:, 96:128])

    w_lin = jnp.dot(h_sum_in, wW_ref[...],
                    preferred_element_type=jnp.float32) + wb_ref[...]
    m32 = jnp.full((HID, HID), 1.0 / HID, dtype=jnp.float32)
    u = jnp.dot(w_lin, m32, preferred_element_type=jnp.float32)
    d = w_lin - u
    sv = jnp.dot(d * d, m32, preferred_element_type=jnp.float32)
    ln_out = d * (wlnw_ref[...] * jax.lax.rsqrt(sv + 1e-05)) + wlnb_ref[...]
    out_ref[0] = h_a + jax.nn.relu(ln_out)


def _blkdiag4(w):
    return jax.scipy.linalg.block_diag(w, w, w, w)


@functools.partial(jax.jit, static_argnames=())
def _run(corr_index, nei_index, hidden_state, dest_corr, agent_v,
         rel_W, rel_b, rel_ln_w, rel_ln_b,
         ng_W, ng_b, ng_ln_w, ng_ln_b,
         war_W, war_b, w_W, w_b, w_ln_w, w_ln_b):
    p = N * N
    corr4 = jnp.concatenate(
        [corr_index.reshape(p, 2), jnp.tile(agent_v, (N, 1))], axis=-1)
    corr4_p = (corr4.reshape(N, G, NB, 4).transpose(0, 2, 1, 3)
               .reshape(P4, 4 * G))
    dc_p = (dest_corr.reshape(S, N, G, NB, 2).transpose(0, 1, 3, 2, 4)
            .reshape(S, P4, 2 * G))

    relW4 = jnp.concatenate([rel_W[0:2], rel_W[4:6]], axis=0)
    w4blk = _blkdiag4(relW4)
    w2blk = _blkdiag4(rel_W[2:4])
    mblk = _blkdiag4(jnp.full((HID, HID), 1.0 / HID, jnp.float32))
    ngWr_blk = _blkdiag4(ng_W[0:HID])
    warWr_blk = _blkdiag4(war_W[0:HID])
    rup = _blkdiag4(jnp.ones((1, HID), jnp.float32))

    t4 = lambda v: jnp.tile(v, G)
    full = lambda shape: pl.BlockSpec(shape, lambda s, ia: (0,) * len(shape))
    grid_spec = pl.GridSpec(
        grid=(S, N // BA),
        in_specs=[
            pl.BlockSpec((PB, 4 * G), lambda s, ia: (ia, 0)),
            pl.BlockSpec((1, PB, 2 * G), lambda s, ia: (s, ia, 0)),
            pl.BlockSpec((BA, N), lambda s, ia: (ia, 0)),
            pl.BlockSpec((1, N, HID), lambda s, ia: (s, 0, 0)),
            full((128, 128)),
            full((16, 128)), full((8, 128)),
            full((128,)), full((128,)), full((128,)),
            full((128, 128)), full((HID, HID)), full((HID, HID)),
            full((128,)), full((128,)), full((128,)),
            full((128, G)), full((HID, 1)), full((HID, 1)), full((1,)),
            full((G, 128)),
            full((HID, HID)), full((HID,)), full((HID,)), full((HID,)),
        ],
        out_specs=pl.BlockSpec((1, BA, HID), lambda s, ia: (s, ia, 0)),
    )
    return pl.pallas_call(
        _body,
        grid_spec=grid_spec,
        compiler_params=pltpu.CompilerParams(
            dimension_semantics=("parallel", "parallel")),
        out_shape=jax.ShapeDtypeStruct((S, N, HID), jnp.float32),
    )(corr4_p, dc_p, nei_index, hidden_state, mblk,
      w4blk, w2blk, t4(rel_b), t4(rel_ln_w), t4(rel_ln_b),
      ngWr_blk, ng_W[HID:2 * HID], ng_W[2 * HID:], t4(ng_b), t4(ng_ln_w),
      t4(ng_ln_b),
      warWr_blk, war_W[HID:2 * HID], war_W[2 * HID:], war_b, rup,
      w_W, w_b, w_ln_w, w_ln_b)


def kernel(corr_index, nei_index, nei_num, hidden_state, dest_corr, past_dest,
           agent_v, rel_W, rel_b, rel_ln_w, rel_ln_b, ng_W, ng_b, ng_ln_w,
           ng_ln_b, war_W, war_b, w_W, w_b, w_ln_w, w_ln_b):
    del nei_num, past_dest
    return _run(corr_index, nei_index, hidden_state, dest_corr, agent_v,
                rel_W, rel_b, rel_ln_w, rel_ln_b,
                ng_W, ng_b, ng_ln_w, ng_ln_b,
                war_W, war_b, w_W, w_b, w_ln_w, w_ln_b)


# speedup vs baseline: 1.2631x; 1.2631x over previous
import functools

import jax
import jax.numpy as jnp
from jax.experimental import pallas as pl
from jax.experimental.pallas import tpu as pltpu

S = 20
N = 128
HID = 32
G = 4
NB = N // G
P4 = N * NB
ST = 2
SA = ST * N
PR = ST * P4


def _split4(x):
    sl = lambda lo, hi: x[..., lo:hi, :]
    return jnp.concatenate(
        [sl(0, 32), sl(32, 64), sl(64, 96), sl(96, 128)], axis=-1)


def _body(corr_ref, dc_ref, ni_ref, hs_ref, mblk_ref,
          w4blk_ref, w2blk_ref, relb_ref, rellnw_ref, rellnb_ref,
          ngWr_ref, ngWh_ref, ngWn_ref, ngb_ref, nglnw_ref, nglnb_ref,
          warWr_ref, warWh_ref, warWn_ref, warb_ref, rup_ref,
          wW_ref, wb_ref, wlnw_ref, wlnb_ref,
          out_ref):
    mblk = mblk_ref[...]

    def ln_packed(x, w, b, eps=1e-05):
        u = jnp.dot(x, mblk, preferred_element_type=jnp.float32)
        d = x - u
        s = jnp.dot(d * d, mblk, preferred_element_type=jnp.float32)
        return d * (w * jax.lax.rsqrt(s + eps)) + b

    h3 = hs_ref[...]
    h_all = h3.reshape(SA, HID)

    corr_part = jnp.dot(corr_ref[...], w4blk_ref[...],
                        preferred_element_type=jnp.float32)
    corr_bc = jnp.broadcast_to(corr_part[None], (ST, P4, 128)).reshape(PR, 128)
    r_lin = (corr_bc
             + jnp.dot(dc_ref[...].reshape(PR, 2 * G), w2blk_ref[...],
                       preferred_element_type=jnp.float32)
             + relb_ref[...])
    r_t = jax.nn.relu(ln_packed(r_lin, rellnw_ref[...], rellnb_ref[...]))

    hh = jnp.dot(h_all, ngWh_ref[...], preferred_element_type=jnp.float32)
    hn = jnp.dot(h_all, ngWn_ref[...], preferred_element_type=jnp.float32)
    hh128 = jnp.concatenate([hh, hh, hh, hh], axis=1)
    hh_bc = jnp.broadcast_to(hh128[:, None, :], (SA, NB, 128)).reshape(PR, 128)
    hn4 = _split4(hn.reshape(ST, N, HID))
    hn_bc = jnp.broadcast_to(hn4[:, None, :, :],
                             (ST, N, NB, 128)).reshape(PR, 128)
    ng_lin = (jnp.dot(r_t, ngWr_ref[...], preferred_element_type=jnp.float32)
              + hh_bc + hn_bc + ngb_ref[...])
    nGate = jax.nn.sigmoid(ln_packed(ng_lin, nglnw_ref[...], nglnb_ref[...]))

    wh = jnp.dot(h_all, warWh_ref[...], preferred_element_type=jnp.float32)
    wn = jnp.dot(h_all, warWn_ref[...], preferred_element_type=jnp.float32)
    wn4 = _split4(wn.reshape(ST, N, 1))
    wn_bc = jnp.broadcast_to(wn4[:, None, :, :],
                             (ST, N, NB, G)).reshape(SA, NB, G)
    z4 = (jnp.dot(r_t, warWr_ref[...], preferred_element_type=jnp.float32)
          .reshape(SA, NB, G)
          + wh.reshape(SA, 1, 1) + wn_bc + warb_ref[...])
    tt3 = jnp.swapaxes(jax.nn.relu(z4), 1, 2)
    tt_nn = jnp.concatenate(
        [tt3[:, 0, :], tt3[:, 1, :], tt3[:, 2, :], tt3[:, 3, :]], axis=1)

    mask = jnp.broadcast_to((ni_ref[...] > 0)[None], (ST, N, N)).reshape(SA, N)
    pos_in = jnp.where(mask & (tt_nn != 0.0), tt_nn, -10000.0)
    pos = jax.nn.softmax(pos_in, axis=-1)
    coef = jnp.where(mask, pos, 0.0)

    c34 = jnp.concatenate(
        [coef[:, 0:32].reshape(SA, 1, NB), coef[:, 32:64].reshape(SA, 1, NB),
         coef[:, 64:96].reshape(SA, 1, NB), coef[:, 96:128].reshape(SA, 1, NB)],
        axis=1)
    coef4 = jnp.swapaxes(c34, 1, 2).reshape(PR, G)
    coef_p = jnp.dot(coef4, rup_ref[...],
                     preferred_element_type=jnp.float32)

    nb = _split4(h3)
    nei_p = jnp.broadcast_to(nb[:, None, :, :],
                             (ST, N, NB, 128)).reshape(PR, 128)
    hm = nei_p * nGate * coef_p
    hsum = hm.reshape(SA, NB, 128).sum(axis=1)
    h_sum_in = (hsum[:, 0:32] + hsum[:, 32:64]
                + hsum[:, 64:96] + hsum[:, 96:128])

    w_lin = jnp.dot(h_sum_in, wW_ref[...],
                    preferred_element_type=jnp.float32) + wb_ref[...]
    m32 = jnp.full((HID, HID), 1.0 / HID, dtype=jnp.float32)
    u = jnp.dot(w_lin, m32, preferred_element_type=jnp.float32)
    d = w_lin - u
    sv = jnp.dot(d * d, m32, preferred_element_type=jnp.float32)
    ln_out = d * (wlnw_ref[...] * jax.lax.rsqrt(sv + 1e-05)) + wlnb_ref[...]
    out_ref[...] = (h_all + jax.nn.relu(ln_out)).reshape(ST, N, HID)


def _blkdiag4(w):
    return jax.scipy.linalg.block_diag(w, w, w, w)


@functools.partial(jax.jit, static_argnames=())
def _run(corr_index, nei_index, hidden_state, dest_corr, agent_v,
         rel_W, rel_b, rel_ln_w, rel_ln_b,
         ng_W, ng_b, ng_ln_w, ng_ln_b,
         war_W, war_b, w_W, w_b, w_ln_w, w_ln_b):
    p = N * N
    corr4 = jnp.concatenate(
        [corr_index.reshape(p, 2), jnp.tile(agent_v, (N, 1))], axis=-1)
    corr4_p = (corr4.reshape(N, G, NB, 4).transpose(0, 2, 1, 3)
               .reshape(P4, 4 * G))
    dc_p = (dest_corr.reshape(S, N, G, NB, 2).transpose(0, 1, 3, 2, 4)
            .reshape(S, P4, 2 * G))

    relW4 = jnp.concatenate([rel_W[0:2], rel_W[4:6]], axis=0)
    w4blk = _blkdiag4(relW4)
    w2blk = _blkdiag4(rel_W[2:4])
    mblk = _blkdiag4(jnp.full((HID, HID), 1.0 / HID, jnp.float32))
    ngWr_blk = _blkdiag4(ng_W[0:HID])
    warWr_blk = _blkdiag4(war_W[0:HID])
    rup = _blkdiag4(jnp.ones((1, HID), jnp.float32))

    t4 = lambda v: jnp.tile(v, G)
    full = lambda shape: pl.BlockSpec(shape, lambda t: (0,) * len(shape))
    grid_spec = pl.GridSpec(
        grid=(S // ST,),
        in_specs=[
            full((P4, 4 * G)),
            pl.BlockSpec((ST, P4, 2 * G), lambda t: (t, 0, 0)),
            full((N, N)),
            pl.BlockSpec((ST, N, HID), lambda t: (t, 0, 0)),
            full((128, 128)),
            full((16, 128)), full((8, 128)),
            full((128,)), full((128,)), full((128,)),
            full((128, 128)), full((HID, HID)), full((HID, HID)),
            full((128,)), full((128,)), full((128,)),
            full((128, G)), full((HID, 1)), full((HID, 1)), full((1,)),
            full((G, 128)),
            full((HID, HID)), full((HID,)), full((HID,)), full((HID,)),
        ],
        out_specs=pl.BlockSpec((ST, N, HID), lambda t: (t, 0, 0)),
    )
    return pl.pallas_call(
        _body,
        grid_spec=grid_spec,
        compiler_params=pltpu.CompilerParams(
            dimension_semantics=("parallel",)),
        out_shape=jax.ShapeDtypeStruct((S, N, HID), jnp.float32),
    )(corr4_p, dc_p, nei_index, hidden_state, mblk,
      w4blk, w2blk, t4(rel_b), t4(rel_ln_w), t4(rel_ln_b),
      ngWr_blk, ng_W[HID:2 * HID], ng_W[2 * HID:], t4(ng_b), t4(ng_ln_w),
      t4(ng_ln_b),
      warWr_blk, war_W[HID:2 * HID], war_W[2 * HID:], war_b, rup,
      w_W, w_b, w_ln_w, w_ln_b)


def kernel(corr_index, nei_index, nei_num, hidden_state, dest_corr, past_dest,
           agent_v, rel_W, rel_b, rel_ln_w, rel_ln_b, ng_W, ng_b, ng_ln_w,
           ng_ln_b, war_W, war_b, w_W, w_b, w_ln_w, w_ln_b):
    del nei_num, past_dest
    return _run(corr_index, nei_index, hidden_state, dest_corr, agent_v,
                rel_W, rel_b, rel_ln_w, rel_ln_b,
                ng_W, ng_b, ng_ln_w, ng_ln_b,
                war_W, war_b, w_W, w_b, w_ln_w, w_ln_b)
